# Initial kernel scaffold; baseline (speedup 1.0000x reference)
import jax
import jax.numpy as jnp
from jax.experimental import pallas as pl


def kernel(input_ids, attention_mask, embed_table):
    raise NotImplementedError("write your pallas kernel here")


# speedup vs baseline: 2.9947x; 2.9947x over previous
import functools

import jax
import jax.numpy as jnp
from jax import lax
from jax.experimental import pallas as pl
from jax.experimental.pallas import tpu as pltpu
from jax.experimental.pallas import tpu_sc as plsc

NC = 2
NS = 16
NW = NC * NS
CHUNK = 128


def kernel(input_ids, attention_mask, embed_table):
    B, S = input_ids.shape
    V, D = embed_table.shape
    N = B * S
    per_w = N // NW
    n_chunks = per_w // CHUNK
    ids = input_ids.reshape(NW, n_chunks, CHUNK).astype(jnp.int32)

    mesh = plsc.VectorSubcoreMesh(core_axis_name="c", subcore_axis_name="s")

    @functools.partial(
        pl.kernel,
        out_type=jax.ShapeDtypeStruct((N, D), jnp.float32),
        mesh=mesh,
        scratch_types=[
            pltpu.VMEM((n_chunks, CHUNK), jnp.int32),
            pltpu.VMEM((CHUNK, D), jnp.float32),
            pltpu.SemaphoreType.DMA,
        ],
        compiler_params=pltpu.CompilerParams(use_tc_tiling_on_sc=False),
    )
    def emb(ids_hbm, table_hbm, out_hbm, idx_v, rows_v, sem):
        wid = lax.axis_index("s") * NC + lax.axis_index("c")
        base = wid * per_w
        pltpu.sync_copy(ids_hbm.at[wid], idx_v)

        def body(j, carry):
            pltpu.async_copy(table_hbm.at[idx_v.at[j]], rows_v, sem).wait()
            pltpu.sync_copy(rows_v, out_hbm.at[pl.ds(base + j * CHUNK, CHUNK)])
            return carry

        lax.fori_loop(0, n_chunks, body, 0)

    out = emb(ids, embed_table)
    return out.reshape(B, S, D)


# speedup vs baseline: 3.0550x; 1.0202x over previous
import functools

import jax
import jax.numpy as jnp
from jax import lax
from jax.experimental import pallas as pl
from jax.experimental.pallas import tpu as pltpu
from jax.experimental.pallas import tpu_sc as plsc

NC = 2
NS = 16
NW = NC * NS
CHUNK = 128


def kernel(input_ids, attention_mask, embed_table):
    B, S = input_ids.shape
    V, D = embed_table.shape
    N = B * S
    per_w = N // NW
    n_chunks = per_w // CHUNK
    ids = input_ids.reshape(NW, n_chunks, CHUNK).astype(jnp.int32)

    mesh = plsc.VectorSubcoreMesh(core_axis_name="c", subcore_axis_name="s")

    GCH = 4
    GROUP = GCH * CHUNK
    n_groups = n_chunks // GCH

    @functools.partial(
        pl.kernel,
        out_type=jax.ShapeDtypeStruct((N, D), jnp.float32),
        mesh=mesh,
        scratch_types=[
            pltpu.VMEM((n_chunks, CHUNK), jnp.int32),
            pltpu.VMEM((2, GROUP, D), jnp.float32),
            pltpu.SemaphoreType.DMA((2,)),
            pltpu.SemaphoreType.DMA((2,)),
        ],
        compiler_params=pltpu.CompilerParams(use_tc_tiling_on_sc=False),
    )
    def emb(ids_hbm, table_hbm, out_hbm, idx_v, rows_v, gsem, wsem):
        wid = lax.axis_index("s") * NC + lax.axis_index("c")
        base = wid * per_w
        pltpu.sync_copy(ids_hbm.at[wid], idx_v)

        def fire_gathers(g, b):
            for k in range(GCH):
                pltpu.async_copy(
                    table_hbm.at[idx_v.at[g * GCH + k]],
                    rows_v.at[b, pl.ds(k * CHUNK, CHUNK)],
                    gsem.at[b],
                )

        def wait_gathers(g, b):
            for k in range(GCH):
                pltpu.make_async_copy(
                    table_hbm.at[idx_v.at[g * GCH + k]],
                    rows_v.at[b, pl.ds(k * CHUNK, CHUNK)],
                    gsem.at[b],
                ).wait()

        def fire_write(g, b):
            pltpu.async_copy(
                rows_v.at[b], out_hbm.at[pl.ds(base + g * GROUP, GROUP)], wsem.at[b]
            )

        def wait_write(b):
            pltpu.make_async_copy(
                rows_v.at[b], out_hbm.at[pl.ds(base, GROUP)], wsem.at[b]
            ).wait()

        fire_gathers(0, 0)

        def body(gg, carry):
            for b in range(2):
                g = gg * 2 + b
                wait_gathers(g, b)
                fire_write(g, b)
                ob = 1 - b

                @pl.when(g + 1 < n_groups)
                def _():
                    @pl.when(g >= 1)
                    def _():
                        wait_write(ob)

                    fire_gathers(g + 1, ob)

            return carry

        lax.fori_loop(0, n_groups // 2, body, 0)
        wait_write(0)
        wait_write(1)

    out = emb(ids, embed_table)
    return out.reshape(B, S, D)


# speedup vs baseline: 5.0119x; 1.6405x over previous
import functools

import jax
import jax.numpy as jnp
from jax import lax
from jax.experimental import pallas as pl
from jax.experimental.pallas import tpu as pltpu
from jax.experimental.pallas import tpu_sc as plsc

NC = 2
NS = 16
NW = NC * NS
CHUNK = 128


def kernel(input_ids, attention_mask, embed_table):
    B, S = input_ids.shape
    V, D = embed_table.shape
    N = B * S
    per_w = N // NW
    n_chunks = per_w // CHUNK
    ids = input_ids.reshape(NW, n_chunks, CHUNK).astype(jnp.int32)

    mesh = plsc.VectorSubcoreMesh(core_axis_name="c", subcore_axis_name="s")

    GCH = 4
    GROUP = GCH * CHUNK
    n_groups = n_chunks // GCH

    @functools.partial(
        pl.kernel,
        out_type=jax.ShapeDtypeStruct((N, D), jnp.float32),
        mesh=mesh,
        scratch_types=[
            pltpu.VMEM((n_chunks, CHUNK), jnp.int32),
            pltpu.VMEM((2, GROUP, D), jnp.float32),
            pltpu.VMEM_SHARED((V, D), jnp.float32),
            pltpu.SemaphoreType.DMA((2,)),
            pltpu.SemaphoreType.DMA((2,)),
        ],
        compiler_params=pltpu.CompilerParams(use_tc_tiling_on_sc=False),
    )
    def emb(ids_hbm, table_hbm, out_hbm, idx_v, rows_v, table_sh, gsem, wsem):
        wid = lax.axis_index("s") * NC + lax.axis_index("c")
        base = wid * per_w
        sid = lax.axis_index("s")

        @pl.when(sid == 0)
        def _():
            pltpu.sync_copy(table_hbm, table_sh)

        pltpu.sync_copy(ids_hbm.at[wid], idx_v)
        plsc.subcore_barrier()

        def fire_gathers(g, b):
            for k in range(GCH):
                pltpu.async_copy(
                    table_sh.at[idx_v.at[g * GCH + k]],
                    rows_v.at[b, pl.ds(k * CHUNK, CHUNK)],
                    gsem.at[b],
                )

        def wait_gathers(g, b):
            for k in range(GCH):
                pltpu.make_async_copy(
                    table_sh.at[idx_v.at[g * GCH + k]],
                    rows_v.at[b, pl.ds(k * CHUNK, CHUNK)],
                    gsem.at[b],
                ).wait()

        def fire_write(g, b):
            pltpu.async_copy(
                rows_v.at[b], out_hbm.at[pl.ds(base + g * GROUP, GROUP)], wsem.at[b]
            )

        def wait_write(b):
            pltpu.make_async_copy(
                rows_v.at[b], out_hbm.at[pl.ds(base, GROUP)], wsem.at[b]
            ).wait()

        fire_gathers(0, 0)

        def body(gg, carry):
            for b in range(2):
                g = gg * 2 + b
                wait_gathers(g, b)
                fire_write(g, b)
                ob = 1 - b

                @pl.when(g + 1 < n_groups)
                def _():
                    @pl.when(g >= 1)
                    def _():
                        wait_write(ob)

                    fire_gathers(g + 1, ob)

            return carry

        lax.fori_loop(0, n_groups // 2, body, 0)
        wait_write(0)
        wait_write(1)

    out = emb(ids, embed_table)
    return out.reshape(B, S, D)
